# baseline (speedup 1.0000x reference)
import jax
import jax.numpy as jnp
from jax import lax
from jax.experimental import pallas as pl
from jax.experimental.pallas import tpu as pltpu
from jax.experimental.pallas import tpu_sc as plsc

N_TOKENS = 8192
NUM_CLASSES = 1024
EMBED_DIM = 128

T_TC = 4096
T_SC = N_TOKENS - T_TC

NC = 2
NS = 16
NW = NC * NS
TOK_PER_W = T_SC // NW
CHUNK = 128
CHUNKS_PER_W = TOK_PER_W // CHUNK
ROWS_PER_S = NUM_CLASSES // NS

A_BLK = 2048
A_STEPS = T_TC // A_BLK
B_BLK = 2048
B_STEPS = N_TOKENS // B_BLK

_PREC = jax.lax.Precision.DEFAULT


def _sc_segsum_kernel(x_hbm, y_hbm, zero_hbm, part_hbm,
                      x_v, idx_v, ones_v, acc_sum, acc_cnt,
                      sem_x, sem_y):
    c = lax.axis_index("c")
    s = lax.axis_index("s")
    wid = s * NC + c

    cp_x = pltpu.async_copy(x_hbm.at[pl.ds(T_TC + wid * TOK_PER_W, TOK_PER_W)],
                            x_v, sem_x)
    cp_y = pltpu.async_copy(y_hbm.at[pl.ds(wid * CHUNKS_PER_W, CHUNKS_PER_W)],
                            idx_v, sem_y)

    def _fill_ones(r, carry):
        for k in range(CHUNK // 16):
            ones_v[r, pl.ds(k * 16, 16)] = jnp.ones((16,), jnp.float32)
        return carry
    lax.fori_loop(0, CHUNK, _fill_ones, 0)

    pltpu.sync_copy(zero_hbm, acc_sum.at[pl.ds(s * ROWS_PER_S, ROWS_PER_S)])
    pltpu.sync_copy(zero_hbm, acc_cnt.at[pl.ds(s * ROWS_PER_S, ROWS_PER_S)])

    cp_x.wait()
    cp_y.wait()
    plsc.subcore_barrier()

    for j in range(CHUNKS_PER_W):
        pltpu.sync_copy(x_v.at[pl.ds(j * CHUNK, CHUNK)],
                        acc_sum.at[idx_v.at[j]], add=True)
        pltpu.sync_copy(ones_v, acc_cnt.at[idx_v.at[j]], add=True)
    plsc.subcore_barrier()

    pltpu.sync_copy(acc_sum.at[pl.ds(s * ROWS_PER_S, ROWS_PER_S)],
                    part_hbm.at[c, pl.ds(s * ROWS_PER_S, ROWS_PER_S)])
    pltpu.sync_copy(acc_cnt.at[pl.ds(s * ROWS_PER_S, ROWS_PER_S)],
                    part_hbm.at[c, pl.ds(NUM_CLASSES + s * ROWS_PER_S,
                                         ROWS_PER_S)])


def _sc_segsum(x, y_sc):
    y2 = y_sc.reshape(NW * CHUNKS_PER_W, CHUNK)
    zero = jnp.zeros((ROWS_PER_S, EMBED_DIM), jnp.float32)
    mesh = plsc.VectorSubcoreMesh(core_axis_name="c", subcore_axis_name="s")
    fn = pl.kernel(
        _sc_segsum_kernel,
        out_type=jax.ShapeDtypeStruct((NC, 2 * NUM_CLASSES, EMBED_DIM),
                                      jnp.float32),
        mesh=mesh,
        scratch_types=[
            pltpu.VMEM((TOK_PER_W, EMBED_DIM), jnp.float32),
            pltpu.VMEM((CHUNKS_PER_W, CHUNK), jnp.int32),
            pltpu.VMEM((CHUNK, EMBED_DIM), jnp.float32),
            pltpu.VMEM_SHARED((NUM_CLASSES, EMBED_DIM), jnp.float32),
            pltpu.VMEM_SHARED((NUM_CLASSES, EMBED_DIM), jnp.float32),
            pltpu.SemaphoreType.DMA,
            pltpu.SemaphoreType.DMA,
        ],
    )
    return fn(x, y2, zero)


def _acc_kernel(x_ref, y_ref, sums_out, cnt_out, sums_ref, cnt_ref):
    i = pl.program_id(0)

    @pl.when(i == 0)
    def _init():
        sums_ref[...] = jnp.zeros_like(sums_ref)
        cnt_ref[...] = jnp.zeros_like(cnt_ref)

    y_blk = y_ref[...]
    cls = jax.lax.broadcasted_iota(jnp.int32, (A_BLK, NUM_CLASSES), 1)
    oh = (y_blk == cls).astype(jnp.float32)
    sums_ref[...] += jax.lax.dot_general(
        oh, x_ref[...], (((0,), (0,)), ((), ())),
        precision=_PREC, preferred_element_type=jnp.float32)
    cnt_ref[...] += jax.lax.dot_general(
        oh, jnp.ones((A_BLK, 8), jnp.float32), (((0,), (0,)), ((), ())),
        precision=_PREC, preferred_element_type=jnp.float32)

    @pl.when(i == A_STEPS - 1)
    def _emit():
        sums_out[...] = sums_ref[...]
        cnt_out[...] = cnt_ref[...]


def _tc_accum(x, y_tc):
    y2 = y_tc.reshape(T_TC, 1)
    return pl.pallas_call(
        _acc_kernel,
        grid=(A_STEPS,),
        in_specs=[
            pl.BlockSpec((A_BLK, EMBED_DIM), lambda i: (i, 0)),
            pl.BlockSpec((A_BLK, 1), lambda i: (i, 0)),
        ],
        out_specs=[
            pl.BlockSpec((NUM_CLASSES, EMBED_DIM), lambda i: (0, 0)),
            pl.BlockSpec((NUM_CLASSES, 8), lambda i: (0, 0)),
        ],
        out_shape=[
            jax.ShapeDtypeStruct((NUM_CLASSES, EMBED_DIM), jnp.float32),
            jax.ShapeDtypeStruct((NUM_CLASSES, 8), jnp.float32),
        ],
        scratch_shapes=[
            pltpu.VMEM((NUM_CLASSES, EMBED_DIM), jnp.float32),
            pltpu.VMEM((NUM_CLASSES, 8), jnp.float32),
        ],
    )(x[:T_TC], y2)


def _dist_kernel(part_ref, tsum_ref, tcnt_ref, p_ref, c_ref, x_ref, o_ref,
                 u2_ref, usq_ref):
    i = pl.program_id(0)

    @pl.when(i == 0)
    def _update():
        sums = (part_ref[0, :NUM_CLASSES] + part_ref[1, :NUM_CLASSES]
                + tsum_ref[...])
        cnt = (part_ref[0, NUM_CLASSES:, 0:1] + part_ref[1, NUM_CLASSES:, 0:1]
               + tcnt_ref[:, 0:1])
        new = sums / jnp.maximum(cnt, 1.0)
        c = c_ref[...]
        u = jnp.where(cnt > 0.0, (c * p_ref[...] + new) / (c + 1.0),
                      p_ref[...])
        u2_ref[...] = u + u
        usq_ref[...] = jax.lax.dot_general(
            jnp.ones((1, EMBED_DIM), jnp.float32), u * u,
            (((1,), (1,)), ((), ())),
            precision=_PREC, preferred_element_type=jnp.float32)

    x = x_ref[...]
    d2 = jax.lax.dot_general(x, u2_ref[...], (((1,), (1,)), ((), ())),
                             precision=_PREC,
                             preferred_element_type=jnp.float32)
    xsq = jax.lax.dot_general(x * x, jnp.ones((1, EMBED_DIM), jnp.float32),
                              (((1,), (1,)), ((), ())),
                              precision=_PREC,
                              preferred_element_type=jnp.float32)
    o_ref[...] = jnp.minimum(d2 - xsq - usq_ref[...], 0.0)


def kernel(x, y_true, prototypes, counter):
    c2 = counter.reshape(NUM_CLASSES, 1)
    part = _sc_segsum(x, y_true[T_TC:])
    tsum, tcnt = _tc_accum(x, y_true[:T_TC])

    out = pl.pallas_call(
        _dist_kernel,
        grid=(B_STEPS,),
        in_specs=[
            pl.BlockSpec((NC, 2 * NUM_CLASSES, EMBED_DIM),
                         lambda i: (0, 0, 0)),
            pl.BlockSpec((NUM_CLASSES, EMBED_DIM), lambda i: (0, 0)),
            pl.BlockSpec((NUM_CLASSES, 8), lambda i: (0, 0)),
            pl.BlockSpec((NUM_CLASSES, EMBED_DIM), lambda i: (0, 0)),
            pl.BlockSpec((NUM_CLASSES, 1), lambda i: (0, 0)),
            pl.BlockSpec((B_BLK, EMBED_DIM), lambda i: (i, 0)),
        ],
        out_specs=pl.BlockSpec((B_BLK, NUM_CLASSES), lambda i: (i, 0)),
        out_shape=jax.ShapeDtypeStruct((N_TOKENS, NUM_CLASSES), jnp.float32),
        scratch_shapes=[
            pltpu.VMEM((NUM_CLASSES, EMBED_DIM), jnp.float32),
            pltpu.VMEM((1, NUM_CLASSES), jnp.float32),
        ],
    )(part, tsum, tcnt, prototypes, c2, x)
    return out


# speedup vs baseline: 1.5324x; 1.5324x over previous
import jax
import jax.numpy as jnp
from jax.experimental import pallas as pl
from jax.experimental.pallas import tpu as pltpu

N_TOKENS = 8192
NUM_CLASSES = 1024
EMBED_DIM = 128

BLK = 4096
STEPS = N_TOKENS // BLK

_PREC = jax.lax.Precision.DEFAULT


def _fused_kernel(x_ref, y_ref, p_ref, c_ref, o_ref, sums_ref, cnt_ref,
                  u2_ref, usq_ref, xsq_ref):
    p = pl.program_id(0)
    i = pl.program_id(1)

    @pl.when((p == 0) & (i == 0))
    def _init():
        sums_ref[...] = jnp.zeros_like(sums_ref)
        cnt_ref[...] = jnp.zeros_like(cnt_ref)

    @pl.when(p == 0)
    def _accum():
        y_blk = y_ref[...]
        cls = jax.lax.broadcasted_iota(jnp.int32, (BLK, NUM_CLASSES), 1)
        oh = (y_blk == cls).astype(jnp.bfloat16)
        x = x_ref[...]
        sums_ref[...] += jax.lax.dot_general(
            oh, x.astype(jnp.bfloat16), (((0,), (0,)), ((), ())),
            precision=_PREC, preferred_element_type=jnp.float32)
        cnt_ref[...] += jax.lax.dot_general(
            oh, jnp.ones((BLK, 8), jnp.bfloat16), (((0,), (0,)), ((), ())),
            precision=_PREC, preferred_element_type=jnp.float32)
        xsq_ref[pl.ds(i * BLK, BLK), :] = jax.lax.dot_general(
            x * x, jnp.ones((1, EMBED_DIM), jnp.float32),
            (((1,), (1,)), ((), ())),
            precision=_PREC, preferred_element_type=jnp.float32)

    @pl.when((p == 0) & (i == STEPS - 1))
    def _update():
        cnt = cnt_ref[:, 0:1]
        new = sums_ref[...] / jnp.maximum(cnt, 1.0)
        c = c_ref[...]
        u = jnp.where(cnt > 0.0, (c * p_ref[...] + new) / (c + 1.0),
                      p_ref[...])
        u2_ref[...] = u + u
        usq_ref[...] = jax.lax.dot_general(
            jnp.ones((1, EMBED_DIM), jnp.float32), u * u,
            (((1,), (1,)), ((), ())),
            precision=_PREC, preferred_element_type=jnp.float32)

    @pl.when(p == 1)
    def _dist():
        d2 = jax.lax.dot_general(x_ref[...], u2_ref[...],
                                 (((1,), (1,)), ((), ())),
                                 precision=_PREC,
                                 preferred_element_type=jnp.float32)
        o_ref[...] = jnp.minimum(
            d2 - xsq_ref[pl.ds(i * BLK, BLK), :] - usq_ref[...], 0.0)


def kernel(x, y_true, prototypes, counter):
    y2 = y_true.reshape(N_TOKENS, 1)
    c2 = counter.reshape(NUM_CLASSES, 1)
    out = pl.pallas_call(
        _fused_kernel,
        grid=(2, STEPS),
        in_specs=[
            pl.BlockSpec((BLK, EMBED_DIM), lambda p, i: (i, 0)),
            pl.BlockSpec((BLK, 1), lambda p, i: (i, 0)),
            pl.BlockSpec((NUM_CLASSES, EMBED_DIM), lambda p, i: (0, 0)),
            pl.BlockSpec((NUM_CLASSES, 1), lambda p, i: (0, 0)),
        ],
        out_specs=pl.BlockSpec((BLK, NUM_CLASSES), lambda p, i: (i * p, 0)),
        out_shape=jax.ShapeDtypeStruct((N_TOKENS, NUM_CLASSES), jnp.float32),
        scratch_shapes=[
            pltpu.VMEM((NUM_CLASSES, EMBED_DIM), jnp.float32),
            pltpu.VMEM((NUM_CLASSES, 8), jnp.float32),
            pltpu.VMEM((NUM_CLASSES, EMBED_DIM), jnp.float32),
            pltpu.VMEM((1, NUM_CLASSES), jnp.float32),
            pltpu.VMEM((N_TOKENS, 1), jnp.float32),
        ],
    )(x, y2, prototypes, c2)
    return out


# speedup vs baseline: 1.7733x; 1.1572x over previous
import jax
import jax.numpy as jnp
from jax.experimental import pallas as pl
from jax.experimental.pallas import tpu as pltpu

N_TOKENS = 8192
NUM_CLASSES = 1024
EMBED_DIM = 128
AUG = EMBED_DIM + 8

BLK = 2048
STEPS = N_TOKENS // BLK

_PREC = jax.lax.Precision.DEFAULT


def _fused_kernel(x_ref, y_ref, p_ref, c_ref, o_ref, acc_ref, ua_ref,
                  xsq_ref):
    p = pl.program_id(0)
    i = pl.program_id(1)

    @pl.when((p == 0) & (i == 0))
    def _init():
        acc_ref[...] = jnp.zeros_like(acc_ref)

    @pl.when(p == 0)
    def _accum():
        y_blk = y_ref[...]
        cls = jax.lax.broadcasted_iota(jnp.int32, (BLK, NUM_CLASSES), 1)
        oh = (y_blk == cls).astype(jnp.bfloat16)
        x = x_ref[...]
        xcat = jnp.concatenate(
            [x.astype(jnp.bfloat16),
             jnp.ones((BLK, AUG - EMBED_DIM), jnp.bfloat16)], axis=1)
        acc_ref[...] += jax.lax.dot_general(
            oh, xcat, (((0,), (0,)), ((), ())),
            precision=_PREC, preferred_element_type=jnp.float32)
        xsq_ref[pl.ds(i * BLK, BLK), :] = jax.lax.dot_general(
            x * x, jnp.ones((1, EMBED_DIM), jnp.float32),
            (((1,), (1,)), ((), ())),
            precision=_PREC, preferred_element_type=jnp.float32)

    @pl.when((p == 0) & (i == STEPS - 1))
    def _update():
        cnt = acc_ref[:, EMBED_DIM:EMBED_DIM + 1]
        sums = acc_ref[:, :EMBED_DIM]
        new = sums / jnp.maximum(cnt, 1.0)
        c = c_ref[...]
        u = jnp.where(cnt > 0.0, (c * p_ref[...] + new) / (c + 1.0),
                      p_ref[...])
        usq = jnp.sum(u * u, axis=1, keepdims=True)
        ua_ref[...] = jnp.concatenate(
            [u + u, jnp.ones((NUM_CLASSES, 1), jnp.float32), -usq,
             jnp.zeros((NUM_CLASSES, AUG - EMBED_DIM - 2), jnp.float32)],
            axis=1).astype(jnp.bfloat16)

    @pl.when(p == 1)
    def _dist():
        x = x_ref[...]
        xa = jnp.concatenate(
            [x.astype(jnp.bfloat16),
             -xsq_ref[pl.ds(i * BLK, BLK), :].astype(jnp.bfloat16),
             jnp.ones((BLK, 1), jnp.bfloat16),
             jnp.zeros((BLK, AUG - EMBED_DIM - 2), jnp.bfloat16)], axis=1)
        d = jax.lax.dot_general(xa, ua_ref[...], (((1,), (1,)), ((), ())),
                                precision=_PREC,
                                preferred_element_type=jnp.float32)
        o_ref[...] = jnp.minimum(d, 0.0)


def kernel(x, y_true, prototypes, counter):
    y2 = y_true.reshape(N_TOKENS, 1)
    c2 = counter.reshape(NUM_CLASSES, 1)
    out = pl.pallas_call(
        _fused_kernel,
        grid=(2, STEPS),
        in_specs=[
            pl.BlockSpec((BLK, EMBED_DIM), lambda p, i: (i, 0)),
            pl.BlockSpec((BLK, 1), lambda p, i: (i, 0)),
            pl.BlockSpec((NUM_CLASSES, EMBED_DIM), lambda p, i: (0, 0)),
            pl.BlockSpec((NUM_CLASSES, 1), lambda p, i: (0, 0)),
        ],
        out_specs=pl.BlockSpec((BLK, NUM_CLASSES), lambda p, i: (i * p, 0)),
        out_shape=jax.ShapeDtypeStruct((N_TOKENS, NUM_CLASSES), jnp.float32),
        scratch_shapes=[
            pltpu.VMEM((NUM_CLASSES, AUG), jnp.float32),
            pltpu.VMEM((NUM_CLASSES, AUG), jnp.bfloat16),
            pltpu.VMEM((N_TOKENS, 1), jnp.float32),
        ],
    )(x, y2, prototypes, c2)
    return out
